# Initial kernel scaffold; baseline (speedup 1.0000x reference)
import jax
import jax.numpy as jnp
from jax.experimental import pallas as pl


def kernel(x, Wr_w, Wr_b, Wl_w, Wl_b, router_w, router_b, W1, b1, W2, b2, style):
    raise NotImplementedError("write your pallas kernel here")


# speedup vs baseline: 1.1120x; 1.1120x over previous
import functools

import jax
import jax.numpy as jnp
from jax import lax
from jax.experimental import pallas as pl
from jax.experimental.pallas import tpu as pltpu
from jax.experimental.pallas import tpu_sc as plsc

N = 4096
DIM = 1024
MARKET = 32
STYLE = 32
E = 8
TOPK = 2
ALPHA = 1e-4

TILE = 256
NT = N * TOPK // TILE + E
P_PAD = NT * TILE

NW = 32

NEG_INF = float("-inf")


K1_BLK = 512


def _k1_body(x_ref, wr_ref, wrb_ref, wl_ref, rx_ref, lx_ref, m_ref, macc):
    i = pl.program_id(0)

    @pl.when(i == 0)
    def _():
        macc[...] = jnp.zeros_like(macc)

    xb = x_ref[...]
    market = jax.nn.relu(
        lax.dot_general(xb, wr_ref[...], (((1,), (1,)), ((), ())),
                        preferred_element_type=jnp.float32) + wrb_ref[...])
    macc[...] += lax.dot_general(wl_ref[...], market, (((1,), (0,)), ((), ())),
                                 preferred_element_type=jnp.float32)
    lx_ref[...] = lax.dot_general(xb, rx_ref[...], (((1,), (1,)), ((), ())),
                                  preferred_element_type=jnp.float32)
    m_ref[...] = macc[...]


def _run_k1(x2d, Wr_w, Wr_b, Wl_w, Rx):
    return pl.pallas_call(
        _k1_body,
        grid=(N // K1_BLK,),
        in_specs=[
            pl.BlockSpec((K1_BLK, DIM), lambda i: (i, 0)),
            pl.BlockSpec((MARKET, DIM), lambda i: (0, 0)),
            pl.BlockSpec((1, MARKET), lambda i: (0, 0)),
            pl.BlockSpec((1, K1_BLK), lambda i: (0, i)),
            pl.BlockSpec((E, DIM), lambda i: (0, 0)),
        ],
        out_specs=[
            pl.BlockSpec((K1_BLK, E), lambda i: (i, 0)),
            pl.BlockSpec((1, MARKET), lambda i: (0, 0)),
        ],
        out_shape=[
            jax.ShapeDtypeStruct((N, E), jnp.float32),
            jax.ShapeDtypeStruct((1, MARKET), jnp.float32),
        ],
        scratch_shapes=[pltpu.VMEM((1, MARKET), jnp.float32)],
    )(x2d, Wr_w, Wr_b.reshape(1, MARKET), Wl_w, Rx)


K2_BLK = 128


def _k2_body(lx_ref, m_ref, rm_ref, rb_ref,
             e1_ref, e2_ref, g1_ref, g2_ref, r1_ref, r2_ref,
             counts_ref, aux_ref, ccnt, cg):
    i = pl.program_id(0)

    @pl.when(i == 0)
    def _():
        ccnt[...] = jnp.zeros_like(ccnt)
        cg[...] = jnp.zeros_like(cg)

    lm = lax.dot_general(m_ref[...], rm_ref[...], (((1,), (1,)), ((), ())),
                         preferred_element_type=jnp.float32)
    logits = lx_ref[...] + lm + rb_ref[...]
    iota = lax.broadcasted_iota(jnp.int32, (K2_BLK, E), 1)
    v1 = jnp.max(logits, axis=1, keepdims=True)
    i1 = jnp.min(jnp.where(logits == v1, iota, E), axis=1, keepdims=True)
    masked = jnp.where(iota == i1, NEG_INF, logits)
    v2 = jnp.max(masked, axis=1, keepdims=True)
    i2 = jnp.min(jnp.where(masked == v2, iota, E), axis=1, keepdims=True)

    z = jnp.exp(v2 - v1)
    g1 = 1.0 / (1.0 + z)
    g2 = z * g1

    hit1 = iota == i1
    hit2 = iota == i2
    hits = (hit1 | hit2).astype(jnp.float32)

    row = lax.broadcasted_iota(jnp.int32, (K2_BLK, K2_BLK), 0)
    col = lax.broadcasted_iota(jnp.int32, (K2_BLK, K2_BLK), 1)
    ltri = (col < row).astype(jnp.float32)
    intra = lax.dot_general(ltri, hits, (((1,), (0,)), ((), ())),
                            preferred_element_type=jnp.float32)
    rank = ccnt[...] + intra
    r1 = jnp.sum(jnp.where(hit1, rank, 0.0), axis=1)
    r2 = jnp.sum(jnp.where(hit2, rank, 0.0), axis=1)

    gate = hit1.astype(jnp.float32) * g1 + hit2.astype(jnp.float32) * g2
    ccnt[...] += jnp.sum(hits, axis=0, keepdims=True)
    cg[...] += jnp.sum(gate, axis=0, keepdims=True)

    e1_ref[...] = i1[:, 0]
    e2_ref[...] = i2[:, 0]
    g1_ref[...] = g1[:, 0]
    g2_ref[...] = g2[:, 0]
    r1_ref[...] = r1.astype(jnp.int32)
    r2_ref[...] = r2.astype(jnp.int32)
    counts_ref[...] = ccnt[...]
    fi = ccnt[...] * (E / (TOPK * N))
    pi = cg[...] * (1.0 / N)
    aux_ref[...] = jnp.sum(fi * pi).reshape(1, 1) * ALPHA


def _run_k2(lx, m, Rm, rb):
    vec = lambda: pl.BlockSpec((K2_BLK,), lambda i: (i,))
    return pl.pallas_call(
        _k2_body,
        grid=(N // K2_BLK,),
        in_specs=[
            pl.BlockSpec((K2_BLK, E), lambda i: (i, 0)),
            pl.BlockSpec((1, MARKET), lambda i: (0, 0)),
            pl.BlockSpec((E, MARKET), lambda i: (0, 0)),
            pl.BlockSpec((1, E), lambda i: (0, 0)),
        ],
        out_specs=[
            vec(), vec(), vec(), vec(), vec(), vec(),
            pl.BlockSpec((1, E), lambda i: (0, 0)),
            pl.BlockSpec((1, 1), lambda i: (0, 0)),
        ],
        out_shape=[
            jax.ShapeDtypeStruct((N,), jnp.int32),
            jax.ShapeDtypeStruct((N,), jnp.int32),
            jax.ShapeDtypeStruct((N,), jnp.float32),
            jax.ShapeDtypeStruct((N,), jnp.float32),
            jax.ShapeDtypeStruct((N,), jnp.int32),
            jax.ShapeDtypeStruct((N,), jnp.int32),
            jax.ShapeDtypeStruct((1, E), jnp.float32),
            jax.ShapeDtypeStruct((1, 1), jnp.float32),
        ],
        scratch_shapes=[pltpu.VMEM((1, E), jnp.float32),
                        pltpu.VMEM((1, E), jnp.float32)],
    )(lx, m, Rm, rb)


K2B_BLK = 512


def _k2b_body(e1_ref, e2_ref, r1_ref, r2_ref, counts_ref,
              d1_ref, d2_ref, tmap_ref):
    cnt_pad = jnp.ceil(counts_ref[...] * (1.0 / TILE)) * TILE
    li = lax.broadcasted_iota(jnp.int32, (E, E), 0)
    lj = lax.broadcasted_iota(jnp.int32, (E, E), 1)
    ltri = (li < lj).astype(jnp.float32)
    offs = lax.dot_general(cnt_pad, ltri, (((1,), (0,)), ((), ())),
                           preferred_element_type=jnp.float32)

    iota = lax.broadcasted_iota(jnp.int32, (K2B_BLK, E), 1)
    e1 = e1_ref[...].reshape(K2B_BLK, 1)
    e2 = e2_ref[...].reshape(K2B_BLK, 1)
    off1 = jnp.sum(jnp.where(iota == e1, offs, 0.0), axis=1)
    off2 = jnp.sum(jnp.where(iota == e2, offs, 0.0), axis=1)
    d1_ref[...] = (off1 + r1_ref[...].astype(jnp.float32)).astype(jnp.int32)
    d2_ref[...] = (off2 + r2_ref[...].astype(jnp.float32)).astype(jnp.int32)

    ts = (TILE * lax.broadcasted_iota(jnp.int32, (NT, 1), 0)).astype(
        jnp.float32)
    tm = jnp.sum((offs <= ts).astype(jnp.int32), axis=1) - 1
    tmap_ref[...] = jnp.clip(tm, 0, E - 1).reshape(1, NT)


def _run_k2b(e1, e2, r1, r2, counts):
    vec = lambda: pl.BlockSpec((K2B_BLK,), lambda i: (i,))
    return pl.pallas_call(
        _k2b_body,
        grid=(N // K2B_BLK,),
        in_specs=[vec(), vec(), vec(), vec(),
                  pl.BlockSpec((1, E), lambda i: (0, 0))],
        out_specs=[vec(), vec(),
                   pl.BlockSpec((1, NT), lambda i: (0, 0))],
        out_shape=[
            jax.ShapeDtypeStruct((N,), jnp.int32),
            jax.ShapeDtypeStruct((N,), jnp.int32),
            jax.ShapeDtypeStruct((1, NT), jnp.int32),
        ],
    )(e1, e2, r1, r2, counts)


SC1_CH = N // NW


def _sc1_body(d1_h, d2_h, g1_h, g2_h,
              rows_h, gates_h,
              d1v, d2v, g1v, g2v, tokv, sem):
    wid = lax.axis_index("s") * 2 + lax.axis_index("c")
    base = wid * SC1_CH
    pltpu.sync_copy(d1_h.at[pl.ds(base, SC1_CH)], d1v)
    pltpu.sync_copy(d2_h.at[pl.ds(base, SC1_CH)], d2v)
    pltpu.sync_copy(g1_h.at[pl.ds(base, SC1_CH)], g1v)
    pltpu.sync_copy(g2_h.at[pl.ds(base, SC1_CH)], g2v)
    for c in range(SC1_CH // 16):
        tokv[pl.ds(c * 16, 16)] = base + c * 16 + lax.iota(jnp.int32, 16)
    cp1 = pltpu.async_copy(tokv, rows_h.at[d1v], sem)
    cp2 = pltpu.async_copy(tokv, rows_h.at[d2v], sem)
    cp3 = pltpu.async_copy(g1v, gates_h.at[d1v], sem)
    cp4 = pltpu.async_copy(g2v, gates_h.at[d2v], sem)
    cp1.wait()
    cp2.wait()
    cp3.wait()
    cp4.wait()


def _run_sc1(d1, d2, g1, g2):
    mesh = plsc.VectorSubcoreMesh(core_axis_name="c", subcore_axis_name="s")
    f = functools.partial(
        pl.kernel,
        out_type=[
            jax.ShapeDtypeStruct((P_PAD,), jnp.int32),
            jax.ShapeDtypeStruct((P_PAD,), jnp.float32),
        ],
        mesh=mesh,
        scratch_types=[
            pltpu.VMEM((SC1_CH,), jnp.int32),
            pltpu.VMEM((SC1_CH,), jnp.int32),
            pltpu.VMEM((SC1_CH,), jnp.float32),
            pltpu.VMEM((SC1_CH,), jnp.float32),
            pltpu.VMEM((SC1_CH,), jnp.int32),
            pltpu.SemaphoreType.DMA,
        ],
    )(_sc1_body)
    return f(d1, d2, g1, g2)


SC2_RPW = P_PAD // NW
SC2_CH = 64


def _sc2_body(rows_h, x_h, xs_h, idxv, buf, sem):
    wid = lax.axis_index("s") * 2 + lax.axis_index("c")
    base = wid * SC2_RPW
    pltpu.sync_copy(rows_h.at[pl.ds(base, SC2_RPW)], idxv)
    for c in range(SC2_RPW // 16):
        sl = pl.ds(c * 16, 16)
        idxv[sl] = jnp.bitwise_and(idxv[sl], N - 1)
    for b in range(SC2_RPW // SC2_CH):
        pltpu.async_copy(x_h.at[idxv.at[pl.ds(b * SC2_CH, SC2_CH)]], buf,
                         sem).wait()
        pltpu.sync_copy(buf, xs_h.at[pl.ds(base + b * SC2_CH, SC2_CH)])


def _run_sc2(rows, x2d):
    mesh = plsc.VectorSubcoreMesh(core_axis_name="c", subcore_axis_name="s")
    f = functools.partial(
        pl.kernel,
        out_type=jax.ShapeDtypeStruct((P_PAD, DIM), jnp.float32),
        mesh=mesh,
        scratch_types=[
            pltpu.VMEM((SC2_RPW,), jnp.int32),
            pltpu.VMEM((SC2_CH, DIM), jnp.float32),
            pltpu.SemaphoreType.DMA,
        ],
    )(_sc2_body)
    return f(rows, x2d)


def _k3s_body(tmap_ref, xs_ref, gs_ref, w1x_ref, w1s_ref, w2_ref,
              b1_ref, b2_ref, sty_ref, ys_ref):
    pre_s = lax.dot_general(sty_ref[0], w1s_ref[0], (((1,), (1,)), ((), ())),
                            preferred_element_type=jnp.float32)
    h = jax.nn.relu(
        lax.dot_general(xs_ref[...], w1x_ref[0], (((1,), (1,)), ((), ())),
                        preferred_element_type=jnp.float32)
        + pre_s + b1_ref[0])
    y = lax.dot_general(h, w2_ref[0], (((1,), (1,)), ((), ())),
                        preferred_element_type=jnp.float32) + b2_ref[0]
    ys_ref[...] = y * gs_ref[...]


def _run_k3_sparse(tmap, xs, gates2d, W1x, W1s, W2, b1, b2, style):
    grid_spec = pltpu.PrefetchScalarGridSpec(
        num_scalar_prefetch=1,
        grid=(NT,),
        in_specs=[
            pl.BlockSpec((TILE, DIM), lambda i, t: (i, 0)),
            pl.BlockSpec((TILE, 1), lambda i, t: (i, 0)),
            pl.BlockSpec((1, DIM, DIM), lambda i, t: (t[i], 0, 0)),
            pl.BlockSpec((1, DIM, STYLE), lambda i, t: (t[i], 0, 0)),
            pl.BlockSpec((1, DIM, DIM), lambda i, t: (t[i], 0, 0)),
            pl.BlockSpec((1, 1, DIM), lambda i, t: (t[i], 0, 0)),
            pl.BlockSpec((1, 1, DIM), lambda i, t: (t[i], 0, 0)),
            pl.BlockSpec((1, 1, STYLE), lambda i, t: (t[i], 0, 0)),
        ],
        out_specs=pl.BlockSpec((TILE, DIM), lambda i, t: (i, 0)),
    )
    return pl.pallas_call(
        _k3s_body,
        grid_spec=grid_spec,
        out_shape=jax.ShapeDtypeStruct((P_PAD, DIM), jnp.float32),
    )(tmap, xs, gates2d, W1x, W1s, W2,
      b1.reshape(E, 1, DIM), b2.reshape(E, 1, DIM),
      style.reshape(E, 1, STYLE))


SC3_CH = N // NW
SC3_SUB = 32


def _sc3_body(d1_h, d2_h, ys_h, out_h, d1v, d2v, bufA, bufB, sem):
    wid = lax.axis_index("s") * 2 + lax.axis_index("c")
    base = wid * SC3_CH
    pltpu.sync_copy(d1_h.at[pl.ds(base, SC3_CH)], d1v)
    pltpu.sync_copy(d2_h.at[pl.ds(base, SC3_CH)], d2v)
    for c in range(SC3_CH // SC3_SUB):
        ca = pltpu.async_copy(ys_h.at[d1v.at[pl.ds(c * SC3_SUB, SC3_SUB)]],
                              bufA, sem)
        cb = pltpu.async_copy(ys_h.at[d2v.at[pl.ds(c * SC3_SUB, SC3_SUB)]],
                              bufB, sem)
        ca.wait()
        cb.wait()
        for t in range(SC3_SUB):
            def add_body(v, _, t=t):
                sl = pl.ds(v * 16, 16)
                bufA[t, sl] = bufA[t, sl] + bufB[t, sl]
                return 0
            lax.fori_loop(0, DIM // 16, add_body, 0, unroll=4)
        pltpu.sync_copy(bufA, out_h.at[pl.ds(base + c * SC3_SUB, SC3_SUB)])


def _run_sc3(d1, d2, ys):
    mesh = plsc.VectorSubcoreMesh(core_axis_name="c", subcore_axis_name="s")
    f = functools.partial(
        pl.kernel,
        out_type=jax.ShapeDtypeStruct((N, DIM), jnp.float32),
        mesh=mesh,
        scratch_types=[
            pltpu.VMEM((SC3_CH,), jnp.int32),
            pltpu.VMEM((SC3_CH,), jnp.int32),
            pltpu.VMEM((SC3_SUB, DIM), jnp.float32),
            pltpu.VMEM((SC3_SUB, DIM), jnp.float32),
            pltpu.SemaphoreType.DMA,
        ],
    )(_sc3_body)
    return f(d1, d2, ys)


def kernel(x, Wr_w, Wr_b, Wl_w, Wl_b, router_w, router_b, W1, b1, W2, b2, style):
    n, s, d = x.shape
    x2d = x.reshape(n * s, d)
    Rx = router_w[:, :DIM]
    Rm = router_w[:, DIM:]

    lx, m = _run_k1(x2d, Wr_w, Wr_b, Wl_w, Rx)
    m = m + Wl_b.reshape(1, 1)

    (e1, e2, g1, g2, r1, r2, counts, aux) = _run_k2(
        lx, m, Rm, router_b.reshape(1, E))

    d1, d2, tmap2d = _run_k2b(e1, e2, r1, r2, counts)
    tmap = tmap2d.reshape(NT)

    rows, gates_s = _run_sc1(d1, d2, g1, g2)
    xs = _run_sc2(rows, x2d)

    W1x = W1[:, :, :DIM]
    W1s = W1[:, :, DIM:]
    ys = _run_k3_sparse(tmap, xs, gates_s.reshape(P_PAD, 1),
                        W1x, W1s, W2, b1, b2, style)

    out = _run_sc3(d1, d2, ys)
    return out.reshape(n, s, d), aux.reshape(())
